# Initial kernel scaffold; baseline (speedup 1.0000x reference)
import jax
import jax.numpy as jnp
from jax.experimental import pallas as pl


def kernel(x, edge_attr, params, edge_index, batch):
    raise NotImplementedError("write your pallas kernel here")


# speedup vs baseline: 3.2012x; 3.2012x over previous
import functools

import jax
import jax.numpy as jnp
from jax import lax
from jax.experimental import pallas as pl
from jax.experimental.pallas import tpu as pltpu
from jax.experimental.pallas import tpu_sc as plsc

_F32 = jnp.float32
_NC, _NS = 2, 16
_NW = _NC * _NS
_SUB = 80


def _mesh():
    return plsc.VectorSubcoreMesh(core_axis_name="c", subcore_axis_name="s")


def _tc_emb(x, w, b, blk=1000, interpret=False):
    m, k = x.shape
    n = w.shape[1]
    def body(x_ref, w_ref, b_ref, o_ref):
        o_ref[...] = (jnp.dot(x_ref[...], w_ref[...],
                              preferred_element_type=_F32) + b_ref[...])
    return pl.pallas_call(
        body,
        grid=(m // blk,),
        in_specs=[pl.BlockSpec((blk, k), lambda i: (i, 0)),
                  pl.BlockSpec((k, n), lambda i: (0, 0)),
                  pl.BlockSpec((1, n), lambda i: (0, 0))],
        out_specs=pl.BlockSpec((blk, n), lambda i: (i, 0)),
        out_shape=jax.ShapeDtypeStruct((m, n), _F32),
        interpret=interpret,
    )(x, w, b.reshape(1, -1))


def _tc_edge_embed(ea_in, w1, b1, w2, b2, blk=4000, interpret=False):
    m, k = ea_in.shape
    n = w2.shape[1]
    def body(e_ref, w1_ref, b1_ref, w2_ref, b2_ref, o_ref):
        t = jax.nn.softplus(jnp.dot(e_ref[...], w1_ref[...],
                                    preferred_element_type=_F32) + b1_ref[...])
        o_ref[...] = jnp.dot(t, w2_ref[...],
                             preferred_element_type=_F32) + b2_ref[...]
    return pl.pallas_call(
        body,
        grid=(m // blk,),
        in_specs=[pl.BlockSpec((blk, k), lambda i: (i, 0)),
                  pl.BlockSpec((k, w1.shape[1]), lambda i: (0, 0)),
                  pl.BlockSpec((1, w1.shape[1]), lambda i: (0, 0)),
                  pl.BlockSpec((w1.shape[1], n), lambda i: (0, 0)),
                  pl.BlockSpec((1, n), lambda i: (0, 0))],
        out_specs=pl.BlockSpec((blk, n), lambda i: (i, 0)),
        out_shape=jax.ShapeDtypeStruct((m, n), _F32),
        interpret=interpret,
    )(ea_in, w1, b1.reshape(1, -1), w2, b2.reshape(1, -1))


def _tc_score(hs, hd, ea, u1, u2, u3, bu, w2rep, s8, s1m, hc,
              blk=1000, interpret=False):
    e_num, dn = hs.shape
    wtot = u1.shape[1]
    hnum = 4

    def body(hd_ref, hs_ref, ea_ref, u1_ref, u2_ref, u3_ref, bu_ref,
             w2_ref, s8_ref, s1_ref, vpe_ref, p_ref):
        y = (jnp.dot(hd_ref[...], u1_ref[...], preferred_element_type=_F32)
             + jnp.dot(hs_ref[...], u2_ref[...], preferred_element_type=_F32)
             + jnp.dot(ea_ref[...], u3_ref[...], preferred_element_type=_F32)
             + bu_ref[...])
        q = y[:, :hc]
        kk = y[:, hc:2 * hc]
        vpe_ref[...] = y[:, 2 * hc:3 * hc]
        pt = jnp.tanh(y[:, 3 * hc:4 * hc])
        ps = jax.nn.softplus(y[:, 4 * hc:5 * hc])
        z = (jnp.dot(q * kk, s8_ref[...], preferred_element_type=_F32)
             + jnp.dot(pt * w2_ref[0, :hc] + ps * w2_ref[0, hc:], s1_ref[...],
                       preferred_element_type=_F32))
        pe = jnp.exp(z * jax.nn.sigmoid(z))
        p_ref[...] = jnp.concatenate(
            [pe, jnp.zeros((pe.shape[0], 16 - hnum), _F32)], axis=1)

    return pl.pallas_call(
        body,
        grid=(e_num // blk,),
        in_specs=[pl.BlockSpec((blk, dn), lambda i: (i, 0)),
                  pl.BlockSpec((blk, dn), lambda i: (i, 0)),
                  pl.BlockSpec((blk, dn), lambda i: (i, 0)),
                  pl.BlockSpec((dn, wtot), lambda i: (0, 0)),
                  pl.BlockSpec((dn, wtot), lambda i: (0, 0)),
                  pl.BlockSpec((dn, wtot), lambda i: (0, 0)),
                  pl.BlockSpec((1, wtot), lambda i: (0, 0)),
                  pl.BlockSpec((1, 2 * hc), lambda i: (0, 0)),
                  pl.BlockSpec((hc, hnum), lambda i: (0, 0)),
                  pl.BlockSpec((hc, hnum), lambda i: (0, 0))],
        out_specs=[pl.BlockSpec((blk, hc), lambda i: (i, 0)),
                   pl.BlockSpec((blk, 16), lambda i: (i, 0))],
        out_shape=[jax.ShapeDtypeStruct((e_num, hc), _F32),
                   jax.ShapeDtypeStruct((e_num, 16), _F32)],
        interpret=interpret,
    )(hd, hs, ea, u1, u2, u3, bu.reshape(1, -1), w2rep.reshape(1, -1), s8, s1m)


def _tc_msg(vpe, alpha, expand, ln_g, ln_b, blk=1000, interpret=False):
    e_num, hc = vpe.shape
    hnum = alpha.shape[1]
    inv_d = 1.0 / hc

    dh = hc // _NC

    def body(v_ref, a_ref, ex_ref, g_ref, b_ref, o_ref):
        rep = jnp.dot(a_ref[...], ex_ref[...], preferred_element_type=_F32)
        m0 = v_ref[...] * rep
        mu = jnp.sum(m0, axis=1, keepdims=True) * inv_d
        var = jnp.sum(m0 * m0, axis=1, keepdims=True) * inv_d - mu * mu
        inv = lax.rsqrt(var + 1e-5)
        msg = (m0 - mu) * inv * g_ref[...] + b_ref[...]
        o_ref[0] = msg[:, :dh]
        o_ref[1] = msg[:, dh:]

    return pl.pallas_call(
        body,
        grid=(e_num // blk,),
        in_specs=[pl.BlockSpec((blk, hc), lambda i: (i, 0)),
                  pl.BlockSpec((blk, hnum), lambda i: (i, 0)),
                  pl.BlockSpec((hnum, hc), lambda i: (0, 0)),
                  pl.BlockSpec((1, hc), lambda i: (0, 0)),
                  pl.BlockSpec((1, hc), lambda i: (0, 0))],
        out_specs=pl.BlockSpec((_NC, blk, dh), lambda i: (0, i, 0)),
        out_shape=jax.ShapeDtypeStruct((_NC, e_num, dh), _F32),
        interpret=interpret,
    )(vpe, alpha, expand, ln_g.reshape(1, -1), ln_b.reshape(1, -1))


def _tc_node_update(agg3, h, wc, bc, bn_g, bn_b, wf, bf, interpret=False):
    _, n, dh = agg3.shape
    hc = _NC * dh
    dn = h.shape[1]
    inv_n = 1.0 / n

    def body(agg_ref, h_ref, wc_ref, bc_ref, g_ref, b_ref, wf_ref, bf_ref,
             o_ref):
        y = (jnp.dot(agg_ref[0], wc_ref[:dh], preferred_element_type=_F32)
             + jnp.dot(agg_ref[1], wc_ref[dh:], preferred_element_type=_F32)
             + bc_ref[...])
        mu = jnp.sum(y, axis=0, keepdims=True) * inv_n
        var = jnp.sum(y * y, axis=0, keepdims=True) * inv_n - mu * mu
        bn = (y - mu) * lax.rsqrt(var + 1e-5) * g_ref[...] + b_ref[...]
        o_ref[...] = (bn * jax.nn.sigmoid(bn)
                      + jnp.dot(h_ref[...], wf_ref[...],
                                preferred_element_type=_F32) + bf_ref[...])

    return pl.pallas_call(
        body,
        in_specs=[pl.BlockSpec((_NC, n, dh), lambda: (0, 0, 0)),
                  pl.BlockSpec((n, dn), lambda: (0, 0)),
                  pl.BlockSpec((hc, dn), lambda: (0, 0)),
                  pl.BlockSpec((1, dn), lambda: (0, 0)),
                  pl.BlockSpec((1, dn), lambda: (0, 0)),
                  pl.BlockSpec((1, dn), lambda: (0, 0)),
                  pl.BlockSpec((dn, dn), lambda: (0, 0)),
                  pl.BlockSpec((1, dn), lambda: (0, 0))],
        out_specs=pl.BlockSpec((n, dn), lambda: (0, 0)),
        out_shape=jax.ShapeDtypeStruct((n, dn), _F32),
        interpret=interpret,
    )(agg3, h, wc, bc.reshape(1, -1), bn_g.reshape(1, -1),
      bn_b.reshape(1, -1), wf, bf.reshape(1, -1))


def _tc_pool(h, batch2d, fc_w, fc_b, out_w, out_b, g_num, interpret=False):
    n, dn = h.shape
    hid = fc_w.shape[1]

    def body(h_ref, b_ref, fcw_ref, fcb_ref, ow_ref, ob_ref, o_ref):
        gids = lax.broadcasted_iota(jnp.int32, (1, g_num), 1)
        oh = (b_ref[...] == gids).astype(_F32)
        s = lax.dot_general(oh, h_ref[...], (((0,), (0,)), ((), ())),
                            preferred_element_type=_F32)
        cnt = jnp.sum(oh, axis=0, keepdims=True)
        pooled = s / jnp.maximum(cnt, 1.0).reshape(g_num, 1)
        fc = jnp.dot(pooled, fcw_ref[...],
                     preferred_element_type=_F32) + fcb_ref[...]
        fc = fc * jax.nn.sigmoid(fc)
        o_ref[...] = jnp.dot(fc, ow_ref[...],
                             preferred_element_type=_F32) + ob_ref[...]

    return pl.pallas_call(
        body,
        in_specs=[pl.BlockSpec((n, dn), lambda: (0, 0)),
                  pl.BlockSpec((n, 1), lambda: (0, 0)),
                  pl.BlockSpec((dn, hid), lambda: (0, 0)),
                  pl.BlockSpec((1, hid), lambda: (0, 0)),
                  pl.BlockSpec((hid, 1), lambda: (0, 0)),
                  pl.BlockSpec((1, 1), lambda: (0, 0))],
        out_specs=pl.BlockSpec((g_num, 1), lambda: (0, 0)),
        out_shape=jax.ShapeDtypeStruct((g_num, 1), _F32),
        interpret=interpret,
    )(h, batch2d, fc_w, fc_b.reshape(1, -1), out_w, out_b.reshape(1, -1))


def _sc_gather(h, src, dst, interpret=False):
    n, d = h.shape
    e_num = src.shape[0]
    ew = e_num // _NW
    ch = 400
    assert ew % ch == 0 and ch % _SUB == 0
    nsub = ch // _SUB

    @functools.partial(
        pl.kernel,
        out_type=(jax.ShapeDtypeStruct((e_num, d), _F32),
                  jax.ShapeDtypeStruct((e_num, d), _F32)),
        mesh=_mesh(),
        compiler_params=pltpu.CompilerParams(use_tc_tiling_on_sc=False, needs_layout_passes=False),
        scratch_types=[pltpu.VMEM((ch,), jnp.int32),
                       pltpu.VMEM((ch, d), _F32),
                       pltpu.SemaphoreType.DMA],
        interpret=interpret,
    )
    def k(h_hbm, src_hbm, dst_hbm, hs_out, hd_out, idx_v, rows_v, sem):
        wid = lax.axis_index("s") * _NC + lax.axis_index("c")

        def chunk(ci, carry):
            b = wid * ew + ci * ch
            for idx_hbm, out_hbm in ((src_hbm, hs_out), (dst_hbm, hd_out)):
                pltpu.sync_copy(idx_hbm.at[pl.ds(b, ch)], idx_v)
                for j in range(nsub):
                    pltpu.async_copy(
                        h_hbm.at[idx_v.at[pl.ds(j * _SUB, _SUB)]],
                        rows_v.at[pl.ds(j * _SUB, _SUB)], sem)
                for j in range(nsub):
                    pltpu.make_async_copy(
                        h_hbm.at[idx_v.at[pl.ds(j * _SUB, _SUB)]],
                        rows_v.at[pl.ds(j * _SUB, _SUB)], sem).wait()
                pltpu.sync_copy(rows_v, out_hbm.at[pl.ds(b, ch)])
            return carry

        lax.fori_loop(0, ew // ch, chunk, 0)

    return k(h, src, dst)


def _sc_segsum(pexp, dst2d, zeros_n4, n, interpret=False):
    e_num = pexp.shape[0]
    ew = e_num // _NS
    ch = 400
    assert ew % ch == 0 and ch % _SUB == 0
    nsub = ch // _SUB
    nrows = n // _NS

    @functools.partial(
        pl.kernel,
        out_type=jax.ShapeDtypeStruct((n, 8), _F32),
        mesh=_mesh(),
        compiler_params=pltpu.CompilerParams(use_tc_tiling_on_sc=False, needs_layout_passes=False),
        scratch_types=[pltpu.VMEM((ch, 16), _F32),
                       pltpu.VMEM((ch // _SUB, _SUB), jnp.int32),
                       pltpu.VMEM_SHARED((n, 16), _F32)],
        interpret=interpret,
    )
    def k(p_hbm, dst_hbm, z_hbm, out_hbm, rect_v, idx_v, sh):
        c = lax.axis_index("c")
        s = lax.axis_index("s")

        @pl.when(c == 0)
        def _work():
            pltpu.sync_copy(z_hbm.at[pl.ds(s * nrows, nrows)],
                            sh.at[pl.ds(s * nrows, nrows)])
            plsc.subcore_barrier()

            def chunk(ci, carry):
                b = s * ew + ci * ch
                pltpu.sync_copy(p_hbm.at[pl.ds(b, ch)], rect_v)
                pltpu.sync_copy(dst_hbm.at[pl.ds(b // _SUB, ch // _SUB)],
                                idx_v)
                for j in range(nsub):
                    pltpu.sync_copy(rect_v.at[pl.ds(j * _SUB, _SUB)],
                                    sh.at[idx_v.at[j]], add=True)
                return carry
            lax.fori_loop(0, ew // ch, chunk, 0)
            plsc.subcore_barrier()

            pltpu.sync_copy(sh.at[pl.ds(s * nrows, nrows), pl.ds(0, 8)],
                            out_hbm.at[pl.ds(s * nrows, nrows)])

    return k(pexp, dst2d, zeros_n4)


def _sc_alpha(p16_flat, dst, asum, n, interpret=False):
    e_num = dst.shape[0]
    ew = e_num // _NW
    ch = 2000
    assert ew % ch == 0 and ch % 16 == 0
    n8 = n * 8

    @functools.partial(
        pl.kernel,
        out_type=jax.ShapeDtypeStruct((e_num * 4,), _F32),
        mesh=_mesh(),
        compiler_params=pltpu.CompilerParams(use_tc_tiling_on_sc=False, needs_layout_passes=False),
        scratch_types=[pltpu.VMEM((n8,), _F32),
                       pltpu.VMEM((ch * 16,), _F32),
                       pltpu.VMEM((ch * 4,), _F32),
                       pltpu.VMEM((ch,), jnp.int32)],
        interpret=interpret,
    )
    def k(p_hbm, dst_hbm, asum_hbm, out_hbm, asv, pv, av, idx_v):
        wid = lax.axis_index("s") * _NC + lax.axis_index("c")
        pltpu.sync_copy(asum_hbm, asv)
        lanes = lax.iota(jnp.int32, 16)

        def chunk(ci, carry):
            b = wid * ew + ci * ch
            pltpu.sync_copy(p_hbm.at[pl.ds(b * 16, ch * 16)], pv)
            pltpu.sync_copy(dst_hbm.at[pl.ds(b, ch)], idx_v)

            def grp(i, c2):
                dstv = idx_v[pl.ds(i * 16, 16)]
                for hh in range(4):
                    vals = plsc.load_gather(pv, [i * 256 + lanes * 16 + hh])
                    sums = plsc.load_gather(asv, [dstv * 8 + hh])
                    plsc.store_scatter(av, [i * 64 + lanes * 4 + hh],
                                       vals / (sums + 1e-16))
                return c2
            lax.fori_loop(0, ch // 16, grp, 0)
            pltpu.sync_copy(av, out_hbm.at[pl.ds(b * 4, ch * 4)])
            return carry
        lax.fori_loop(0, ew // ch, chunk, 0)

    return k(p16_flat, dst, asum)


def _sc_scatter(msg3, dst2d, zeros_nd, n, interpret=False):
    _, e_num, dh = msg3.shape
    ew = e_num // _NS
    ch = 160
    assert ew % ch == 0 and ch % _SUB == 0
    nsub = ch // _SUB
    nrows = n // _NS

    @functools.partial(
        pl.kernel,
        out_type=jax.ShapeDtypeStruct((_NC, n, dh), _F32),
        mesh=_mesh(),
        compiler_params=pltpu.CompilerParams(use_tc_tiling_on_sc=False, needs_layout_passes=False),
        scratch_types=[pltpu.VMEM((ch, dh), _F32),
                       pltpu.VMEM((ch // _SUB, _SUB), jnp.int32),
                       pltpu.VMEM_SHARED((n, dh), _F32)],
        interpret=interpret,
    )
    def k(msg_hbm, dst_hbm, z_hbm, out_hbm, rect_v, idx_v, sh):
        c = lax.axis_index("c")
        s = lax.axis_index("s")

        pltpu.sync_copy(z_hbm.at[pl.ds(s * nrows, nrows)],
                        sh.at[pl.ds(s * nrows, nrows)])
        plsc.subcore_barrier()

        def chunk(ci, carry):
            b = s * ew + ci * ch
            pltpu.sync_copy(msg_hbm.at[c, pl.ds(b, ch)], rect_v)
            pltpu.sync_copy(dst_hbm.at[pl.ds(b // _SUB, ch // _SUB)], idx_v)
            for j in range(nsub):
                pltpu.sync_copy(rect_v.at[pl.ds(j * _SUB, _SUB)],
                                sh.at[idx_v.at[j]], add=True)
            return carry
        lax.fori_loop(0, ew // ch, chunk, 0)
        plsc.subcore_barrier()

        pltpu.sync_copy(sh.at[pl.ds(s * nrows, nrows)],
                        out_hbm.at[c, pl.ds(s * nrows, nrows)])

    return k(msg3, dst2d, zeros_nd)


def _fold_weights(p, dn, hnum, cdim):
    hc = hnum * cdim
    wq, wk, wv, we = p['wq'], p['wk'], p['wv'], p['we']
    bq, bk, bv, be = p['bq'], p['bk'], p['bv'], p['be']
    wa1, ba1 = p['wa1'], p['ba1']
    wb1, bb1 = p['wb1'], p['bb1']
    a1, a2, a3 = wa1[:cdim], wa1[cdim:2 * cdim], wa1[2 * cdim:]
    b1, b2, b3 = wb1[:cdim], wb1[cdim:2 * cdim], wb1[2 * cdim:]
    a23 = a2 + a3
    b23 = b2 + b3

    def heads(w):
        return [w[:, i * cdim:(i + 1) * cdim] for i in range(hnum)]

    wq_h, wk_h, we_h = heads(wq), heads(wk), heads(we)
    zero = jnp.zeros((dn, hc), _F32)
    u1 = jnp.concatenate(
        [wq, zero, zero]
        + [wq_h[i] @ a1 for i in range(hnum)]
        + [wq_h[i] @ b1 for i in range(hnum)], axis=1)
    u2 = jnp.concatenate(
        [jnp.zeros((dn, hc), _F32), wk, wv]
        + [wk_h[i] @ a2 for i in range(hnum)]
        + [wk_h[i] @ b2 for i in range(hnum)], axis=1)
    u3 = jnp.concatenate(
        [jnp.zeros((dn, hc), _F32), we, we]
        + [we_h[i] @ a23 for i in range(hnum)]
        + [we_h[i] @ b23 for i in range(hnum)], axis=1)

    def bh(v):
        return [v[i * cdim:(i + 1) * cdim] for i in range(hnum)]

    bq_h, bk_h, be_h = bh(bq), bh(bk), bh(be)
    pre_a = jnp.concatenate(
        [bq_h[i] @ a1 + bk_h[i] @ a2 + be_h[i] @ a23 + ba1
         for i in range(hnum)])
    pre_b = jnp.concatenate(
        [bq_h[i] @ b1 + bk_h[i] @ b2 + be_h[i] @ b23 + bb1
         for i in range(hnum)])
    bu = jnp.concatenate([bq, bk + be, bv + be, pre_a, pre_b])

    w2rep = jnp.concatenate([jnp.tile(p['wa2'][:, 0], hnum),
                             jnp.tile(p['wb2'][:, 0], hnum)])
    eye = jnp.repeat(jnp.eye(hnum, dtype=_F32), cdim, axis=0)
    s8 = eye * 0.125
    expand = eye.T
    return u1, u2, u3, bu, w2rep, s8, eye, expand


def kernel(x, edge_attr, params, edge_index, batch):
    n, _ = x.shape
    e_num = edge_attr.shape[0]
    g_num = 64
    dn = params['emb_w'].shape[1]
    hc = params['convs'][0]['wq'].shape[1]
    hnum = 4
    cdim = hc // hnum

    src = edge_index[0].astype(jnp.int32)
    dst = edge_index[1].astype(jnp.int32)
    dst2d = dst.reshape(e_num // _SUB, _SUB)
    zeros_n16 = jnp.zeros((n, 16), _F32)
    zeros_nd = jnp.zeros((n, hc // _NC), _F32)

    h = _tc_emb(x, params['emb_w'], params['emb_b'])
    ea = _tc_edge_embed(edge_attr, params['ee_w1'], params['ee_b1'],
                        params['ee_w2'], params['ee_b2'])

    for p in params['convs']:
        u1, u2, u3, bu, w2rep, s8, s1m, expand = _fold_weights(
            p, dn, hnum, cdim)
        hs, hd = _sc_gather(h, src, dst)
        vpe, pexp = _tc_score(hs, hd, ea, u1, u2, u3, bu, w2rep, s8, s1m, hc)
        asum = _sc_segsum(pexp, dst2d, zeros_n16, n)
        alpha = _sc_alpha(pexp.reshape(-1), dst, asum.reshape(-1), n
                          ).reshape(e_num, hnum)
        msg = _tc_msg(vpe, alpha, expand, p['ln_g'], p['ln_b'])
        agg = _sc_scatter(msg, dst2d, zeros_nd, n)
        h = _tc_node_update(agg, h, p['wc'], p['bc'], p['bn_g'], p['bn_b'],
                            p['wf'], p['bf'])

    return _tc_pool(h, batch.astype(jnp.int32).reshape(n, 1),
                    params['fc_w'], params['fc_b'],
                    params['out_w'], params['out_b'], g_num)


# speedup vs baseline: 3.2759x; 1.0233x over previous
import functools

import jax
import jax.numpy as jnp
from jax import lax
from jax.experimental import pallas as pl
from jax.experimental.pallas import tpu as pltpu
from jax.experimental.pallas import tpu_sc as plsc

_F32 = jnp.float32
_NC, _NS = 2, 16
_NW = _NC * _NS
_SUB = 80


def _mesh():
    return plsc.VectorSubcoreMesh(core_axis_name="c", subcore_axis_name="s")


def _tc_emb(x, w, b, blk=1000, interpret=False):
    m, k = x.shape
    n = w.shape[1]
    def body(x_ref, w_ref, b_ref, o_ref):
        o_ref[...] = (jnp.dot(x_ref[...], w_ref[...],
                              preferred_element_type=_F32) + b_ref[...])
    return pl.pallas_call(
        body,
        grid=(m // blk,),
        in_specs=[pl.BlockSpec((blk, k), lambda i: (i, 0)),
                  pl.BlockSpec((k, n), lambda i: (0, 0)),
                  pl.BlockSpec((1, n), lambda i: (0, 0))],
        out_specs=pl.BlockSpec((blk, n), lambda i: (i, 0)),
        out_shape=jax.ShapeDtypeStruct((m, n), _F32),
        interpret=interpret,
    )(x, w, b.reshape(1, -1))


def _tc_edge_embed(ea_in, w1, b1, w2, b2, blk=4000, interpret=False):
    m, k = ea_in.shape
    n = w2.shape[1]
    def body(e_ref, w1_ref, b1_ref, w2_ref, b2_ref, o_ref):
        t = jax.nn.softplus(jnp.dot(e_ref[...], w1_ref[...],
                                    preferred_element_type=_F32) + b1_ref[...])
        o_ref[...] = jnp.dot(t, w2_ref[...],
                             preferred_element_type=_F32) + b2_ref[...]
    return pl.pallas_call(
        body,
        grid=(m // blk,),
        in_specs=[pl.BlockSpec((blk, k), lambda i: (i, 0)),
                  pl.BlockSpec((k, w1.shape[1]), lambda i: (0, 0)),
                  pl.BlockSpec((1, w1.shape[1]), lambda i: (0, 0)),
                  pl.BlockSpec((w1.shape[1], n), lambda i: (0, 0)),
                  pl.BlockSpec((1, n), lambda i: (0, 0))],
        out_specs=pl.BlockSpec((blk, n), lambda i: (i, 0)),
        out_shape=jax.ShapeDtypeStruct((m, n), _F32),
        interpret=interpret,
    )(ea_in, w1, b1.reshape(1, -1), w2, b2.reshape(1, -1))


def _tc_score(hs, hd, ea, u1, u2, u3, bu, w2rep, s8, s1m, hc,
              blk=1000, interpret=False):
    e_num, dn = hs.shape
    wtot = u1.shape[1]
    hnum = 4

    def body(hd_ref, hs_ref, ea_ref, u1_ref, u2_ref, u3_ref, bu_ref,
             w2_ref, s8_ref, s1_ref, vpe_ref, p_ref):
        y = (jnp.dot(hd_ref[...], u1_ref[...], preferred_element_type=_F32)
             + jnp.dot(hs_ref[...], u2_ref[...], preferred_element_type=_F32)
             + jnp.dot(ea_ref[...], u3_ref[...], preferred_element_type=_F32)
             + bu_ref[...])
        q = y[:, :hc]
        kk = y[:, hc:2 * hc]
        vpe_ref[...] = y[:, 2 * hc:3 * hc]
        pt = jnp.tanh(y[:, 3 * hc:4 * hc])
        ps = jax.nn.softplus(y[:, 4 * hc:5 * hc])
        z = (jnp.dot(q * kk, s8_ref[...], preferred_element_type=_F32)
             + jnp.dot(pt * w2_ref[0, :hc] + ps * w2_ref[0, hc:], s1_ref[...],
                       preferred_element_type=_F32))
        pe = jnp.exp(z * jax.nn.sigmoid(z))
        p_ref[...] = jnp.concatenate(
            [pe, jnp.zeros((pe.shape[0], 16 - hnum), _F32)], axis=1)

    return pl.pallas_call(
        body,
        grid=(e_num // blk,),
        in_specs=[pl.BlockSpec((blk, dn), lambda i: (i, 0)),
                  pl.BlockSpec((blk, dn), lambda i: (i, 0)),
                  pl.BlockSpec((blk, dn), lambda i: (i, 0)),
                  pl.BlockSpec((dn, wtot), lambda i: (0, 0)),
                  pl.BlockSpec((dn, wtot), lambda i: (0, 0)),
                  pl.BlockSpec((dn, wtot), lambda i: (0, 0)),
                  pl.BlockSpec((1, wtot), lambda i: (0, 0)),
                  pl.BlockSpec((1, 2 * hc), lambda i: (0, 0)),
                  pl.BlockSpec((hc, hnum), lambda i: (0, 0)),
                  pl.BlockSpec((hc, hnum), lambda i: (0, 0))],
        out_specs=[pl.BlockSpec((blk, hc), lambda i: (i, 0)),
                   pl.BlockSpec((blk, 16), lambda i: (i, 0))],
        out_shape=[jax.ShapeDtypeStruct((e_num, hc), _F32),
                   jax.ShapeDtypeStruct((e_num, 16), _F32)],
        interpret=interpret,
    )(hd, hs, ea, u1, u2, u3, bu.reshape(1, -1), w2rep.reshape(1, -1), s8, s1m)


def _tc_msg(vpe, alpha, expand, ln_g, ln_b, blk=1000, interpret=False):
    e_num, hc = vpe.shape
    hnum = alpha.shape[1]
    inv_d = 1.0 / hc

    dh = hc // _NC

    def body(v_ref, a_ref, ex_ref, g_ref, b_ref, o_ref):
        rep = jnp.dot(a_ref[...], ex_ref[...], preferred_element_type=_F32)
        m0 = v_ref[...] * rep
        mu = jnp.sum(m0, axis=1, keepdims=True) * inv_d
        var = jnp.sum(m0 * m0, axis=1, keepdims=True) * inv_d - mu * mu
        inv = lax.rsqrt(var + 1e-5)
        msg = (m0 - mu) * inv * g_ref[...] + b_ref[...]
        o_ref[0] = msg[:, :dh]
        o_ref[1] = msg[:, dh:]

    return pl.pallas_call(
        body,
        grid=(e_num // blk,),
        in_specs=[pl.BlockSpec((blk, hc), lambda i: (i, 0)),
                  pl.BlockSpec((blk, hnum), lambda i: (i, 0)),
                  pl.BlockSpec((hnum, hc), lambda i: (0, 0)),
                  pl.BlockSpec((1, hc), lambda i: (0, 0)),
                  pl.BlockSpec((1, hc), lambda i: (0, 0))],
        out_specs=pl.BlockSpec((_NC, blk, dh), lambda i: (0, i, 0)),
        out_shape=jax.ShapeDtypeStruct((_NC, e_num, dh), _F32),
        interpret=interpret,
    )(vpe, alpha, expand, ln_g.reshape(1, -1), ln_b.reshape(1, -1))


def _tc_node_update(agg3, h, wc, bc, bn_g, bn_b, wf, bf, interpret=False):
    _, n, dh = agg3.shape
    hc = _NC * dh
    dn = h.shape[1]
    inv_n = 1.0 / n

    def body(agg_ref, h_ref, wc_ref, bc_ref, g_ref, b_ref, wf_ref, bf_ref,
             o_ref):
        y = (jnp.dot(agg_ref[0], wc_ref[:dh], preferred_element_type=_F32)
             + jnp.dot(agg_ref[1], wc_ref[dh:], preferred_element_type=_F32)
             + bc_ref[...])
        mu = jnp.sum(y, axis=0, keepdims=True) * inv_n
        var = jnp.sum(y * y, axis=0, keepdims=True) * inv_n - mu * mu
        bn = (y - mu) * lax.rsqrt(var + 1e-5) * g_ref[...] + b_ref[...]
        o_ref[...] = (bn * jax.nn.sigmoid(bn)
                      + jnp.dot(h_ref[...], wf_ref[...],
                                preferred_element_type=_F32) + bf_ref[...])

    return pl.pallas_call(
        body,
        in_specs=[pl.BlockSpec((_NC, n, dh), lambda: (0, 0, 0)),
                  pl.BlockSpec((n, dn), lambda: (0, 0)),
                  pl.BlockSpec((hc, dn), lambda: (0, 0)),
                  pl.BlockSpec((1, dn), lambda: (0, 0)),
                  pl.BlockSpec((1, dn), lambda: (0, 0)),
                  pl.BlockSpec((1, dn), lambda: (0, 0)),
                  pl.BlockSpec((dn, dn), lambda: (0, 0)),
                  pl.BlockSpec((1, dn), lambda: (0, 0))],
        out_specs=pl.BlockSpec((n, dn), lambda: (0, 0)),
        out_shape=jax.ShapeDtypeStruct((n, dn), _F32),
        interpret=interpret,
    )(agg3, h, wc, bc.reshape(1, -1), bn_g.reshape(1, -1),
      bn_b.reshape(1, -1), wf, bf.reshape(1, -1))


def _tc_pool(h, batch2d, fc_w, fc_b, out_w, out_b, g_num, interpret=False):
    n, dn = h.shape
    hid = fc_w.shape[1]

    def body(h_ref, b_ref, fcw_ref, fcb_ref, ow_ref, ob_ref, o_ref):
        gids = lax.broadcasted_iota(jnp.int32, (1, g_num), 1)
        oh = (b_ref[...] == gids).astype(_F32)
        s = lax.dot_general(oh, h_ref[...], (((0,), (0,)), ((), ())),
                            preferred_element_type=_F32)
        cnt = jnp.sum(oh, axis=0, keepdims=True)
        pooled = s / jnp.maximum(cnt, 1.0).reshape(g_num, 1)
        fc = jnp.dot(pooled, fcw_ref[...],
                     preferred_element_type=_F32) + fcb_ref[...]
        fc = fc * jax.nn.sigmoid(fc)
        o_ref[...] = jnp.dot(fc, ow_ref[...],
                             preferred_element_type=_F32) + ob_ref[...]

    return pl.pallas_call(
        body,
        in_specs=[pl.BlockSpec((n, dn), lambda: (0, 0)),
                  pl.BlockSpec((n, 1), lambda: (0, 0)),
                  pl.BlockSpec((dn, hid), lambda: (0, 0)),
                  pl.BlockSpec((1, hid), lambda: (0, 0)),
                  pl.BlockSpec((hid, 1), lambda: (0, 0)),
                  pl.BlockSpec((1, 1), lambda: (0, 0))],
        out_specs=pl.BlockSpec((g_num, 1), lambda: (0, 0)),
        out_shape=jax.ShapeDtypeStruct((g_num, 1), _F32),
        interpret=interpret,
    )(h, batch2d, fc_w, fc_b.reshape(1, -1), out_w, out_b.reshape(1, -1))


def _sc_gather(h, src, dst, interpret=False):
    n, d = h.shape
    e_num = src.shape[0]
    ew = e_num // _NW
    ch = 400
    assert ew % ch == 0 and ch % _SUB == 0
    nsub = ch // _SUB

    @functools.partial(
        pl.kernel,
        out_type=(jax.ShapeDtypeStruct((e_num, d), _F32),
                  jax.ShapeDtypeStruct((e_num, d), _F32)),
        mesh=_mesh(),
        compiler_params=pltpu.CompilerParams(use_tc_tiling_on_sc=False, needs_layout_passes=False),
        scratch_types=[pltpu.VMEM((ch,), jnp.int32),
                       pltpu.VMEM((ch, d), _F32),
                       pltpu.SemaphoreType.DMA],
        interpret=interpret,
    )
    def k(h_hbm, src_hbm, dst_hbm, hs_out, hd_out, idx_v, rows_v, sem):
        wid = lax.axis_index("s") * _NC + lax.axis_index("c")

        def chunk(ci, carry):
            b = wid * ew + ci * ch
            for idx_hbm, out_hbm in ((src_hbm, hs_out), (dst_hbm, hd_out)):
                pltpu.sync_copy(idx_hbm.at[pl.ds(b, ch)], idx_v)
                for j in range(nsub):
                    pltpu.async_copy(
                        h_hbm.at[idx_v.at[pl.ds(j * _SUB, _SUB)]],
                        rows_v.at[pl.ds(j * _SUB, _SUB)], sem)
                for j in range(nsub):
                    pltpu.make_async_copy(
                        h_hbm.at[idx_v.at[pl.ds(j * _SUB, _SUB)]],
                        rows_v.at[pl.ds(j * _SUB, _SUB)], sem).wait()
                pltpu.sync_copy(rows_v, out_hbm.at[pl.ds(b, ch)])
            return carry

        lax.fori_loop(0, ew // ch, chunk, 0)

    return k(h, src, dst)


def _sc_segsum(pexp, dst2d, zeros_n4, n, interpret=False):
    e_num = pexp.shape[0]
    ew = e_num // _NS
    ch = 400
    assert ew % ch == 0 and ch % _SUB == 0
    nsub = ch // _SUB
    nrows = n // _NS

    @functools.partial(
        pl.kernel,
        out_type=jax.ShapeDtypeStruct((n, 8), _F32),
        mesh=_mesh(),
        compiler_params=pltpu.CompilerParams(use_tc_tiling_on_sc=False, needs_layout_passes=False),
        scratch_types=[pltpu.VMEM((ch, 16), _F32),
                       pltpu.VMEM((ch // _SUB, _SUB), jnp.int32),
                       pltpu.VMEM_SHARED((n, 16), _F32)],
        interpret=interpret,
    )
    def k(p_hbm, dst_hbm, z_hbm, out_hbm, rect_v, idx_v, sh):
        c = lax.axis_index("c")
        s = lax.axis_index("s")

        @pl.when(c == 0)
        def _work():
            pltpu.sync_copy(z_hbm.at[pl.ds(s * nrows, nrows)],
                            sh.at[pl.ds(s * nrows, nrows)])
            plsc.subcore_barrier()

            def chunk(ci, carry):
                b = s * ew + ci * ch
                pltpu.sync_copy(p_hbm.at[pl.ds(b, ch)], rect_v)
                pltpu.sync_copy(dst_hbm.at[pl.ds(b // _SUB, ch // _SUB)],
                                idx_v)
                for j in range(nsub):
                    pltpu.sync_copy(rect_v.at[pl.ds(j * _SUB, _SUB)],
                                    sh.at[idx_v.at[j]], add=True)
                return carry
            lax.fori_loop(0, ew // ch, chunk, 0)
            plsc.subcore_barrier()

            pltpu.sync_copy(sh.at[pl.ds(s * nrows, nrows), pl.ds(0, 8)],
                            out_hbm.at[pl.ds(s * nrows, nrows)])

    return k(pexp, dst2d, zeros_n4)


def _sc_alpha(pexp, dst, asum, n, interpret=False):
    e_num = dst.shape[0]
    ew = e_num // _NW
    ch = 2000
    assert ew % ch == 0 and ch % 16 == 0

    @functools.partial(
        pl.kernel,
        out_type=jax.ShapeDtypeStruct((e_num, 4), _F32),
        mesh=_mesh(),
        compiler_params=pltpu.CompilerParams(use_tc_tiling_on_sc=False, needs_layout_passes=False),
        scratch_types=[pltpu.VMEM((n, 8), _F32),
                       pltpu.VMEM((ch, 16), _F32),
                       pltpu.VMEM((ch, 4), _F32),
                       pltpu.VMEM((ch,), jnp.int32)],
        interpret=interpret,
    )
    def k(p_hbm, dst_hbm, asum_hbm, out_hbm, asv, pv, av, idx_v):
        wid = lax.axis_index("s") * _NC + lax.axis_index("c")
        pltpu.sync_copy(asum_hbm, asv)
        lanes = lax.iota(jnp.int32, 16)

        def chunk(ci, carry):
            b = wid * ew + ci * ch
            pltpu.sync_copy(p_hbm.at[pl.ds(b, ch)], pv)
            pltpu.sync_copy(dst_hbm.at[pl.ds(b, ch)], idx_v)

            def grp(i, c2):
                rows = i * 16 + lanes
                dstv = idx_v[pl.ds(i * 16, 16)]
                for hh in range(4):
                    hvec = jnp.full((16,), hh, jnp.int32)
                    vals = plsc.load_gather(pv, [rows, hvec])
                    sums = plsc.load_gather(asv, [dstv, hvec])
                    plsc.store_scatter(av, [rows, hvec],
                                       vals / (sums + 1e-16))
                return c2
            lax.fori_loop(0, ch // 16, grp, 0)
            pltpu.sync_copy(av, out_hbm.at[pl.ds(b, ch)])
            return carry
        lax.fori_loop(0, ew // ch, chunk, 0)

    return k(pexp, dst, asum)


def _sc_scatter(msg3, dst2d, zeros_nd, n, interpret=False):
    _, e_num, dh = msg3.shape
    ew = e_num // _NS
    ch = 160
    assert ew % ch == 0 and ch % _SUB == 0
    nsub = ch // _SUB
    nrows = n // _NS

    @functools.partial(
        pl.kernel,
        out_type=jax.ShapeDtypeStruct((_NC, n, dh), _F32),
        mesh=_mesh(),
        compiler_params=pltpu.CompilerParams(use_tc_tiling_on_sc=False, needs_layout_passes=False),
        scratch_types=[pltpu.VMEM((ch, dh), _F32),
                       pltpu.VMEM((ch // _SUB, _SUB), jnp.int32),
                       pltpu.VMEM_SHARED((n, dh), _F32)],
        interpret=interpret,
    )
    def k(msg_hbm, dst_hbm, z_hbm, out_hbm, rect_v, idx_v, sh):
        c = lax.axis_index("c")
        s = lax.axis_index("s")

        pltpu.sync_copy(z_hbm.at[pl.ds(s * nrows, nrows)],
                        sh.at[pl.ds(s * nrows, nrows)])
        plsc.subcore_barrier()

        def chunk(ci, carry):
            b = s * ew + ci * ch
            pltpu.sync_copy(msg_hbm.at[c, pl.ds(b, ch)], rect_v)
            pltpu.sync_copy(dst_hbm.at[pl.ds(b // _SUB, ch // _SUB)], idx_v)
            for j in range(nsub):
                pltpu.sync_copy(rect_v.at[pl.ds(j * _SUB, _SUB)],
                                sh.at[idx_v.at[j]], add=True)
            return carry
        lax.fori_loop(0, ew // ch, chunk, 0)
        plsc.subcore_barrier()

        pltpu.sync_copy(sh.at[pl.ds(s * nrows, nrows)],
                        out_hbm.at[c, pl.ds(s * nrows, nrows)])

    return k(msg3, dst2d, zeros_nd)


def _fold_weights(p, dn, hnum, cdim):
    hc = hnum * cdim
    wq, wk, wv, we = p['wq'], p['wk'], p['wv'], p['we']
    bq, bk, bv, be = p['bq'], p['bk'], p['bv'], p['be']
    wa1, ba1 = p['wa1'], p['ba1']
    wb1, bb1 = p['wb1'], p['bb1']
    a1, a2, a3 = wa1[:cdim], wa1[cdim:2 * cdim], wa1[2 * cdim:]
    b1, b2, b3 = wb1[:cdim], wb1[cdim:2 * cdim], wb1[2 * cdim:]
    a23 = a2 + a3
    b23 = b2 + b3

    def heads(w):
        return [w[:, i * cdim:(i + 1) * cdim] for i in range(hnum)]

    wq_h, wk_h, we_h = heads(wq), heads(wk), heads(we)
    zero = jnp.zeros((dn, hc), _F32)
    u1 = jnp.concatenate(
        [wq, zero, zero]
        + [wq_h[i] @ a1 for i in range(hnum)]
        + [wq_h[i] @ b1 for i in range(hnum)], axis=1)
    u2 = jnp.concatenate(
        [jnp.zeros((dn, hc), _F32), wk, wv]
        + [wk_h[i] @ a2 for i in range(hnum)]
        + [wk_h[i] @ b2 for i in range(hnum)], axis=1)
    u3 = jnp.concatenate(
        [jnp.zeros((dn, hc), _F32), we, we]
        + [we_h[i] @ a23 for i in range(hnum)]
        + [we_h[i] @ b23 for i in range(hnum)], axis=1)

    def bh(v):
        return [v[i * cdim:(i + 1) * cdim] for i in range(hnum)]

    bq_h, bk_h, be_h = bh(bq), bh(bk), bh(be)
    pre_a = jnp.concatenate(
        [bq_h[i] @ a1 + bk_h[i] @ a2 + be_h[i] @ a23 + ba1
         for i in range(hnum)])
    pre_b = jnp.concatenate(
        [bq_h[i] @ b1 + bk_h[i] @ b2 + be_h[i] @ b23 + bb1
         for i in range(hnum)])
    bu = jnp.concatenate([bq, bk + be, bv + be, pre_a, pre_b])

    w2rep = jnp.concatenate([jnp.tile(p['wa2'][:, 0], hnum),
                             jnp.tile(p['wb2'][:, 0], hnum)])
    eye = jnp.repeat(jnp.eye(hnum, dtype=_F32), cdim, axis=0)
    s8 = eye * 0.125
    expand = eye.T
    return u1, u2, u3, bu, w2rep, s8, eye, expand


def kernel(x, edge_attr, params, edge_index, batch):
    n, _ = x.shape
    e_num = edge_attr.shape[0]
    g_num = 64
    dn = params['emb_w'].shape[1]
    hc = params['convs'][0]['wq'].shape[1]
    hnum = 4
    cdim = hc // hnum

    src = edge_index[0].astype(jnp.int32)
    dst = edge_index[1].astype(jnp.int32)
    dst2d = dst.reshape(e_num // _SUB, _SUB)
    zeros_n16 = jnp.zeros((n, 16), _F32)
    zeros_nd = jnp.zeros((n, hc // _NC), _F32)

    h = _tc_emb(x, params['emb_w'], params['emb_b'])
    ea = _tc_edge_embed(edge_attr, params['ee_w1'], params['ee_b1'],
                        params['ee_w2'], params['ee_b2'])

    for p in params['convs']:
        u1, u2, u3, bu, w2rep, s8, s1m, expand = _fold_weights(
            p, dn, hnum, cdim)
        hs, hd = _sc_gather(h, src, dst)
        vpe, pexp = _tc_score(hs, hd, ea, u1, u2, u3, bu, w2rep, s8, s1m, hc)
        asum = _sc_segsum(pexp, dst2d, zeros_n16, n)
        alpha = _sc_alpha(pexp, dst, asum, n)
        msg = _tc_msg(vpe, alpha, expand, p['ln_g'], p['ln_b'])
        agg = _sc_scatter(msg, dst2d, zeros_nd, n)
        h = _tc_node_update(agg, h, p['wc'], p['bc'], p['bn_g'], p['bn_b'],
                            p['wf'], p['bf'])

    return _tc_pool(h, batch.astype(jnp.int32).reshape(n, 1),
                    params['fc_w'], params['fc_b'],
                    params['out_w'], params['out_b'], g_num)
